# Initial kernel scaffold; baseline (speedup 1.0000x reference)
import jax
import jax.numpy as jnp
from jax.experimental import pallas as pl


def kernel(logits, labels):
    raise NotImplementedError("write your pallas kernel here")


# speedup vs baseline: 1.0542x; 1.0542x over previous
import numpy as np
import jax
import jax.numpy as jnp
from jax.experimental import pallas as pl
from jax.experimental.pallas import tpu as pltpu

_N = 524288
_C = 100
_N_BINS = 15
_BN = 8192
_NBLK = _N // _BN

_BOUNDS = np.linspace(0.0, 1.0, _N_BINS + 1).astype(np.float32)
_LO16 = np.concatenate([_BOUNDS[:-1], np.float32([2.0])])
_HI16 = np.concatenate([_BOUNDS[1:], np.float32([3.0])])


def _ece_kernel(x_ref, lab_ref, bounds_ref, ece_ref, acc_ref, stats_ref):
    i = pl.program_id(0)
    x = x_ref[...]
    lab = lab_ref[0, 0, :]

    m = jnp.max(x, axis=1, keepdims=True)
    s = jnp.sum(jnp.exp(x - m), axis=1)
    conf = 1.0 / s
    pred = jnp.argmax(x, axis=1)
    accv = (pred == lab).astype(jnp.float32)

    lo = bounds_ref[0, :]
    hi = bounds_ref[1, :]
    confc = conf[:, None]
    mask = ((confc > lo[None, :]) & (confc <= hi[None, :])).astype(jnp.float32)
    cnt = jnp.sum(mask, axis=0, keepdims=True)
    sc = jnp.sum(mask * confc, axis=0, keepdims=True)
    sa = jnp.sum(mask * accv[:, None], axis=0, keepdims=True)
    part = jnp.concatenate([cnt, sc, sa], axis=0)

    @pl.when(i == 0)
    def _():
        stats_ref[...] = part

    @pl.when(i > 0)
    def _():
        stats_ref[...] += part

    @pl.when(i == _NBLK - 1)
    def _():
        st = stats_ref[...]
        cntf = st[0, :]
        scf = st[1, :]
        saf = st[2, :]
        safe = jnp.where(cntf > 0, cntf, 1.0)
        prop = cntf * (1.0 / _N)
        avg_acc = saf / safe
        avg_conf = scf / safe
        valid = (cntf > 0).astype(jnp.float32)
        ece = jnp.sum(jnp.abs(avg_conf - avg_acc) * prop * valid) * 100.0
        acc = jnp.sum(avg_acc * prop * valid) * 100.0
        ece_ref[...] = ece.reshape(1, 1)
        acc_ref[...] = acc.reshape(1, 1)


def kernel(logits, labels):
    lab3 = labels.reshape(_NBLK, 1, _BN)
    bounds = jnp.asarray(np.stack([_LO16, _HI16]))
    ece, acc = pl.pallas_call(
        _ece_kernel,
        grid=(_NBLK,),
        in_specs=[
            pl.BlockSpec((_BN, _C), lambda i: (i, 0)),
            pl.BlockSpec((1, 1, _BN), lambda i: (i, 0, 0)),
            pl.BlockSpec((2, 16), lambda i: (0, 0)),
        ],
        out_specs=[
            pl.BlockSpec((1, 1), lambda i: (0, 0)),
            pl.BlockSpec((1, 1), lambda i: (0, 0)),
        ],
        out_shape=[
            jax.ShapeDtypeStruct((1, 1), jnp.float32),
            jax.ShapeDtypeStruct((1, 1), jnp.float32),
        ],
        scratch_shapes=[pltpu.VMEM((3, 16), jnp.float32)],
        compiler_params=pltpu.CompilerParams(
            dimension_semantics=("arbitrary",),
        ),
    )(logits, lab3, bounds)
    return (ece.reshape(1), acc.reshape(1))
